# baseline (speedup 1.0000x reference)
import functools

import jax
import jax.numpy as jnp
from jax import lax
from jax.experimental import pallas as pl
from jax.experimental.pallas import tpu as pltpu
from jax.experimental.pallas import tpu_sc as plsc

N_FACTORS = 64
BATCH = 16384
NC, NS, L = 2, 16, 16
NW = NC * NS
B_PER_W = BATCH // NW
CHUNK = 128
N_CHUNKS = B_PER_W // CHUNK
GROUPS = B_PER_W // L


def _body(user_hbm, item_hbm, uemb_hbm, iemb_hbm, out_hbm,
          uidx_v, iidx_v, urows_v, irows_v, out_v, tscr_v, sem):
  wid = lax.axis_index("s") * NC + lax.axis_index("c")
  base = wid * B_PER_W

  for c in range(N_CHUNKS):
    pltpu.sync_copy(user_hbm.at[pl.ds(base + c * CHUNK, CHUNK)], uidx_v.at[c])
    pltpu.sync_copy(item_hbm.at[pl.ds(base + c * CHUNK, CHUNK)], iidx_v.at[c])

  copies = []
  for c in range(N_CHUNKS):
    copies.append(pltpu.async_copy(
        uemb_hbm.at[uidx_v.at[c]], urows_v.at[pl.ds(c * CHUNK, CHUNK)], sem))
    copies.append(pltpu.async_copy(
        iemb_hbm.at[iidx_v.at[c]], irows_v.at[pl.ds(c * CHUNK, CHUNK)], sem))
  for cp in copies:
    cp.wait()

  iota16 = lax.iota(jnp.int32, L) * L

  def group(g, _):
    for j in range(L):
      r = g * L + j
      partial = jnp.zeros((L,), jnp.float32)
      for k in range(N_FACTORS // L):
        cu = urows_v[r, pl.ds(k * L, L)]
        ci = irows_v[r, pl.ds(k * L, L)]
        partial = partial + cu * ci
      tscr_v[pl.ds(j * L, L)] = partial
    acc = jnp.zeros((L,), jnp.float32)
    for k in range(L):
      acc = acc + plsc.load_gather(tscr_v, [iota16 + k])
    out_v[pl.ds(g * L, L)] = 1.0 / (1.0 + jnp.exp(-acc))
    return 0

  lax.fori_loop(0, GROUPS, group, 0)

  pltpu.sync_copy(out_v, out_hbm.at[pl.ds(base, B_PER_W)])


@jax.jit
def kernel(user, item, user_embed, item_embed):
  mesh = plsc.VectorSubcoreMesh(core_axis_name="c", subcore_axis_name="s")
  run = pl.kernel(
      _body,
      out_type=jax.ShapeDtypeStruct((BATCH,), jnp.float32),
      mesh=mesh,
      compiler_params=pltpu.CompilerParams(
          needs_layout_passes=False, use_tc_tiling_on_sc=False),
      scratch_types=[
          pltpu.VMEM((N_CHUNKS, CHUNK), jnp.int32),
          pltpu.VMEM((N_CHUNKS, CHUNK), jnp.int32),
          pltpu.VMEM((B_PER_W, N_FACTORS), jnp.float32),
          pltpu.VMEM((B_PER_W, N_FACTORS), jnp.float32),
          pltpu.VMEM((B_PER_W,), jnp.float32),
          pltpu.VMEM((L * L,), jnp.float32),
          pltpu.SemaphoreType.DMA,
      ],
  )
  return run(user, item, user_embed, item_embed)
